# Initial kernel scaffold; baseline (speedup 1.0000x reference)
import jax
import jax.numpy as jnp
from jax.experimental import pallas as pl


def kernel(x, message_edge_index, query_edge_index, query_edge_attr, Wl1, Wr1, att1, b1, Wl2, Wr2, att2, b2, Wc1, bc1, Wc2, bc2, Wc3, bc3):
    raise NotImplementedError("write your pallas kernel here")


# speedup vs baseline: 9.0360x; 9.0360x over previous
import functools

import jax
import jax.numpy as jnp
from jax import lax
from jax.experimental import pallas as pl
from jax.experimental.pallas import tpu as pltpu
from jax.experimental.pallas import tpu_sc as plsc

F32 = jnp.float32
I32 = jnp.int32

_N = 10000
_NP = 10112
_RPT = _NP // 16
_DR = 1280
_DRPT = _DR // 16
_N2R = 5120
_N2RPT = _N2R // 16
_E_MSG = 160000
_E2 = _E_MSG + _N
_B = 64
_E2P = 172032
_CH1 = _E2P // (16 * _B)
_CH2 = _E2P // (32 * _B)
_EQ = 160000
_BQ = 128
_CH3 = 40
_EQP = 32 * _BQ * _CH3

_mesh = plsc.VectorSubcoreMesh(core_axis_name="c", subcore_axis_name="s",
                               num_cores=2, num_subcores=16)


def _zero_rows(vbuf, nrows):
    def body(i, _):
        for g in range(8):
            vbuf[i, pl.ds(16 * g, 16)] = jnp.zeros((16,), F32)
        return 0
    lax.fori_loop(0, nrows, body, 0)


def _zero_acc(vbuf, acc, r0, rpt):
    full = rpt // _B
    rem = rpt - full * _B
    for k in range(full):
        pltpu.sync_copy(vbuf, acc.at[pl.ds(r0 + _B * k, _B)])
    if rem:
        pltpu.sync_copy(vbuf.at[pl.ds(0, rem)],
                        acc.at[pl.ds(r0 + full * _B, rem)])


@functools.partial(
    pl.kernel,
    out_type=[jax.ShapeDtypeStruct((2 * _NP, 128), F32),
              jax.ShapeDtypeStruct((2 * _DR, 128), F32)],
    mesh=_mesh,
    compiler_params=pltpu.CompilerParams(needs_layout_passes=False),
    scratch_types=[
        pltpu.VMEM((_B,), I32),
        pltpu.VMEM((_B,), I32),
        pltpu.VMEM((_B,), I32),
        pltpu.VMEM((_B,), I32),
        pltpu.VMEM((_B,), I32),
        pltpu.VMEM((_B,), I32),
        pltpu.VMEM((_B, 128), F32),
        pltpu.VMEM((_B, 128), F32),
        pltpu.VMEM((_B, 128), F32),
        pltpu.VMEM((128,), F32),
        pltpu.VMEM_SHARED((_NP, 128), F32),
        pltpu.VMEM_SHARED((_DR, 128), F32),
        pltpu.SemaphoreType.DMA,
        pltpu.SemaphoreType.DMA,
    ],
)
def _sc_layer1(src_h, dst_h, wdst_h, xl_h, xr_h, att_h, num_o, den_o,
               si, di, wi, wrow, gi, gj, xs, xd, pv, attv,
               nacc, dacc, sem1, sem2):
    cid = lax.axis_index("c")
    sid = lax.axis_index("s")
    r0 = sid * _RPT
    d0 = sid * _DRPT

    _zero_rows(xs, _B)
    _zero_acc(xs, nacc, r0, _RPT)
    _zero_acc(xs, dacc, d0, _DRPT)
    pltpu.sync_copy(att_h.at[cid], attv)
    plsc.subcore_barrier()

    iot = lax.iota(I32, 16)
    off = cid * _NP

    def chunk(c, _):
        ebase = (sid * _CH1 + c) * _B
        pltpu.sync_copy(src_h.at[pl.ds(ebase, _B)], si)
        pltpu.sync_copy(dst_h.at[pl.ds(ebase, _B)], di)
        pltpu.sync_copy(wdst_h.at[pl.ds(ebase, _B)], wi)
        for g in range(_B // 16):
            slc = pl.ds(16 * g, 16)
            gi[slc] = si[slc] + off
            gj[slc] = di[slc] + off
            wrow[slc] = lax.shift_right_logical(wi[slc], 3)
        cp1 = pltpu.async_copy(xl_h.at[gi], xs, sem1)
        cp2 = pltpu.async_copy(xr_h.at[gj], xd, sem2)
        cp1.wait()
        cp2.wait()

        def egrp(go, _):
            wg = wi[pl.ds(16 * go, 16)]
            for e16 in range(16):
                e = 16 * go + e16
                pbs = []
                for lh in range(2):
                    acc = jnp.zeros((16,), F32)
                    for g in range(4):
                        slc = pl.ds(64 * lh + 16 * g, 16)
                        s = xs[e, slc] + xd[e, slc]
                        s = jnp.where(s >= 0.0, s, 0.2 * s)
                        acc = acc + s * attv[pl.ds(64 * lh + 16 * g, 16)]
                    l = jnp.sum(acc)
                    pb = jnp.exp(jnp.broadcast_to(l, (16,)))
                    pbs.append(pb)
                    for g in range(4):
                        slc = pl.ds(64 * lh + 16 * g, 16)
                        xs[e, slc] = xs[e, slc] * pb
                prow = jnp.where(iot == 0, pbs[0],
                                 jnp.where(iot == 1, pbs[1],
                                           jnp.zeros((16,), F32)))
                for g in range(8):
                    pv[e, pl.ds(16 * g, 16)] = jnp.zeros((16,), F32)
                gsel = wg[e16] & 7
                pv[e, pl.ds(16 * gsel, 16)] = prow
            return 0
        lax.fori_loop(0, _B // 16, egrp, 0)
        pltpu.sync_copy(xs, nacc.at[wi], add=True)
        pltpu.sync_copy(pv, dacc.at[wrow], add=True)
        return 0
    lax.fori_loop(0, _CH1, chunk, 0)
    plsc.subcore_barrier()
    pltpu.sync_copy(nacc.at[pl.ds(r0, _RPT)],
                    num_o.at[pl.ds(cid * _NP + r0, _RPT)])
    pltpu.sync_copy(dacc.at[pl.ds(d0, _DRPT)],
                    den_o.at[pl.ds(cid * _DR + d0, _DRPT)])


@functools.partial(
    pl.kernel,
    out_type=[jax.ShapeDtypeStruct((2 * _N2R, 128), F32),
              jax.ShapeDtypeStruct((2 * _DR, 128), F32)],
    mesh=_mesh,
    compiler_params=pltpu.CompilerParams(needs_layout_passes=False),
    scratch_types=[
        pltpu.VMEM((_B,), I32),
        pltpu.VMEM((_B,), I32),
        pltpu.VMEM((_B,), I32),
        pltpu.VMEM((_B,), I32),
        pltpu.VMEM((_B,), I32),
        pltpu.VMEM((_B, 128), F32),
        pltpu.VMEM((_B, 128), F32),
        pltpu.VMEM((_B, 128), F32),
        pltpu.VMEM((_B, 128), F32),
        pltpu.VMEM((64,), F32),
        pltpu.VMEM_SHARED((_N2R, 128), F32),
        pltpu.VMEM_SHARED((_DR, 128), F32),
        pltpu.SemaphoreType.DMA,
        pltpu.SemaphoreType.DMA,
    ],
)
def _sc_layer2(src_h, dst_h, wdst_h, xz_h, att_h, num_o, den_o,
               si, di, wi, wrow, wrow2, xzs, xzd, ys, pv, attv,
               nacc, dacc, sem1, sem2):
    cid = lax.axis_index("c")
    sid = lax.axis_index("s")
    wid = sid * 2 + cid
    r0 = sid * _N2RPT
    d0 = sid * _DRPT

    _zero_rows(ys, _B)
    _zero_acc(ys, nacc, r0, _N2RPT)
    _zero_acc(ys, dacc, d0, _DRPT)
    pltpu.sync_copy(att_h, attv)
    plsc.subcore_barrier()

    iot = lax.iota(I32, 16)

    def chunk(c, _):
        ebase = (wid * _CH2 + c) * _B
        pltpu.sync_copy(src_h.at[pl.ds(ebase, _B)], si)
        pltpu.sync_copy(dst_h.at[pl.ds(ebase, _B)], di)
        pltpu.sync_copy(wdst_h.at[pl.ds(ebase, _B)], wi)
        for g in range(_B // 16):
            slc = pl.ds(16 * g, 16)
            wrow[slc] = lax.shift_right_logical(wi[slc], 3)
            wrow2[slc] = lax.shift_right_logical(wi[slc], 1)
        cp1 = pltpu.async_copy(xz_h.at[si], xzs, sem1)
        cp2 = pltpu.async_copy(xz_h.at[di], xzd, sem2)
        cp1.wait()
        cp2.wait()

        def egrp(go, _):
            wg = wi[pl.ds(16 * go, 16)]
            for e16 in range(16):
                e = 16 * go + e16
                acc = jnp.zeros((16,), F32)
                for g in range(4):
                    s = (xzs[e, pl.ds(16 * g, 16)]
                         + xzd[e, pl.ds(64 + 16 * g, 16)])
                    s = jnp.where(s >= 0.0, s, 0.2 * s)
                    acc = acc + s * attv[pl.ds(16 * g, 16)]
                l = jnp.sum(acc)
                pb = jnp.exp(jnp.broadcast_to(l, (16,)))
                half = 64 * (wg[e16] & 1)
                for g in range(4):
                    ys[e, pl.ds(16 * g, 16)] = jnp.zeros((16,), F32)
                    ys[e, pl.ds(64 + 16 * g, 16)] = jnp.zeros((16,), F32)
                for g in range(4):
                    ys[e, pl.ds(half + 16 * g, 16)] = (
                        xzs[e, pl.ds(16 * g, 16)] * pb)
                prow = jnp.where(iot == 0, pb, jnp.zeros((16,), F32))
                for g in range(8):
                    pv[e, pl.ds(16 * g, 16)] = jnp.zeros((16,), F32)
                gsel = wg[e16] & 7
                pv[e, pl.ds(16 * gsel, 16)] = prow
            return 0
        lax.fori_loop(0, _B // 16, egrp, 0)
        pltpu.sync_copy(ys, nacc.at[wrow2], add=True)
        pltpu.sync_copy(pv, dacc.at[wrow], add=True)
        return 0
    lax.fori_loop(0, _CH2, chunk, 0)
    plsc.subcore_barrier()
    pltpu.sync_copy(nacc.at[pl.ds(r0, _N2RPT)],
                    num_o.at[pl.ds(cid * _N2R + r0, _N2RPT)])
    pltpu.sync_copy(dacc.at[pl.ds(d0, _DRPT)],
                    den_o.at[pl.ds(cid * _DR + d0, _DRPT)])


@functools.partial(
    pl.kernel,
    out_type=[jax.ShapeDtypeStruct((_EQP, 128), F32),
              jax.ShapeDtypeStruct((_EQP, 128), F32)],
    mesh=_mesh,
    compiler_params=pltpu.CompilerParams(needs_layout_passes=False),
    scratch_types=[
        pltpu.VMEM((_BQ,), I32),
        pltpu.VMEM((_BQ,), I32),
        pltpu.VMEM((_BQ, 128), F32),
        pltpu.VMEM((_BQ, 128), F32),
        pltpu.SemaphoreType.DMA,
        pltpu.SemaphoreType.DMA,
    ],
)
def _sc_qgather(qs_h, qd_h, h2_h, hs_o, hd_o, si, di, xs, xd, sem1, sem2):
    cid = lax.axis_index("c")
    sid = lax.axis_index("s")
    wid = sid * 2 + cid

    def chunk(c, _):
        ebase = (wid * _CH3 + c) * _BQ
        pltpu.sync_copy(qs_h.at[pl.ds(ebase, _BQ)], si)
        pltpu.sync_copy(qd_h.at[pl.ds(ebase, _BQ)], di)
        cp1 = pltpu.async_copy(h2_h.at[si], xs, sem1)
        cp2 = pltpu.async_copy(h2_h.at[di], xd, sem2)
        cp1.wait()
        cp2.wait()
        pltpu.sync_copy(xs, hs_o.at[pl.ds(ebase, _BQ)])
        pltpu.sync_copy(xd, hd_o.at[pl.ds(ebase, _BQ)])
        return 0
    lax.fori_loop(0, _CH3, chunk, 0)


def _k1_body(x_ref, w_ref, xl_ref, xr_ref):
    y = jnp.dot(x_ref[...], w_ref[...], preferred_element_type=F32)
    xl_ref[0] = y[:, 0:128]
    xl_ref[1] = y[:, 128:256]
    xr_ref[0] = y[:, 256:384]
    xr_ref[1] = y[:, 384:512]


def _elu(h):
    return jnp.where(h > 0.0, h, jnp.exp(jnp.minimum(h, 0.0)) - 1.0)


def _k3_body(num_ref, den_ref, b1_ref, wl_ref, wr_ref, xz_ref):
    n0 = num_ref[0]
    n1 = num_ref[1]
    d0 = den_ref[0]
    d1 = den_ref[1]
    h = jnp.concatenate([
        n0[:, 0:64] / d0[:, 0:1],
        n0[:, 64:128] / d0[:, 1:2],
        n1[:, 0:64] / d1[:, 0:1],
        n1[:, 64:128] / d1[:, 1:2],
    ], axis=1) + b1_ref[...]
    h = _elu(h)
    xz_ref[:, 0:64] = jnp.dot(h, wl_ref[...], preferred_element_type=F32)
    xz_ref[:, 64:128] = jnp.dot(h, wr_ref[...], preferred_element_type=F32)


def _k5_body(num_ref, den_ref, b2_ref, h2_ref):
    n = num_ref[0] + num_ref[1]
    dd = den_ref[0][:, 0:1] + den_ref[1][:, 0:1]
    h2 = _elu(n / dd + b2_ref[...])
    h2_ref[:, 0:64] = h2
    h2_ref[:, 64:128] = h2


def _k7_body(hs_ref, hd_ref, qa_ref, w1a_ref, w1b_ref, w1c_ref, b1_ref,
             w2_ref, b2_ref, w3_ref, b3_ref, o_ref):
    z = (jnp.dot(hs_ref[...][:, 0:64], w1a_ref[...],
                 preferred_element_type=F32)
         + jnp.dot(hd_ref[...][:, 64:128], w1b_ref[...],
                   preferred_element_type=F32)
         + jnp.dot(qa_ref[...], w1c_ref[...], preferred_element_type=F32)
         + b1_ref[...])
    z = jnp.maximum(z, 0.0)
    z = jnp.dot(z, w2_ref[...], preferred_element_type=F32) + b2_ref[...]
    z = jnp.maximum(z, 0.0)
    o_ref[...] = jnp.dot(z, w3_ref[...], preferred_element_type=F32) + b3_ref[0, 0]


def kernel(x, message_edge_index, query_edge_index, query_edge_attr,
           Wl1, Wr1, att1, b1, Wl2, Wr2, att2, b2,
           Wc1, bc1, Wc2, bc2, Wc3, bc3):
    loop = jnp.arange(_N, dtype=I32)
    pad = _E2P - _E2
    src2 = jnp.concatenate([message_edge_index[0], loop,
                            jnp.zeros((pad,), I32)])
    dst2 = jnp.concatenate([message_edge_index[1], loop,
                            jnp.zeros((pad,), I32)])
    wdst = jnp.concatenate([message_edge_index[1], loop,
                            jnp.full((pad,), _N, I32)])
    qpad = _EQP - _EQ
    qs = jnp.concatenate([query_edge_index[0], jnp.zeros((qpad,), I32)])
    qd = jnp.concatenate([query_edge_index[1], jnp.zeros((qpad,), I32)])
    qa = jnp.concatenate([query_edge_attr, jnp.zeros((qpad, 16), F32)])
    wcat = jnp.concatenate([Wl1, Wr1], axis=1)
    att_cat = att1.reshape(2, 128)

    bn = 1024
    grid_n = (_NP + bn - 1) // bn

    xl_st, xr_st = pl.pallas_call(
        _k1_body,
        grid=(grid_n,),
        in_specs=[pl.BlockSpec((bn, 128), lambda i: (i, 0)),
                  pl.BlockSpec((128, 512), lambda i: (0, 0))],
        out_specs=[pl.BlockSpec((2, bn, 128), lambda i: (0, i, 0)),
                   pl.BlockSpec((2, bn, 128), lambda i: (0, i, 0))],
        out_shape=[jax.ShapeDtypeStruct((2, _NP, 128), F32),
                   jax.ShapeDtypeStruct((2, _NP, 128), F32)],
    )(x, wcat)

    num1, den1 = _sc_layer1(src2, dst2, wdst,
                            xl_st.reshape(2 * _NP, 128),
                            xr_st.reshape(2 * _NP, 128), att_cat)
    den1 = den1.reshape(2, _DR * 8, 16)[:, :_NP, :]

    xz = pl.pallas_call(
        _k3_body,
        grid=(grid_n,),
        in_specs=[pl.BlockSpec((2, bn, 128), lambda i: (0, i, 0)),
                  pl.BlockSpec((2, bn, 16), lambda i: (0, i, 0)),
                  pl.BlockSpec((1, 256), lambda i: (0, 0)),
                  pl.BlockSpec((256, 64), lambda i: (0, 0)),
                  pl.BlockSpec((256, 64), lambda i: (0, 0))],
        out_specs=pl.BlockSpec((bn, 128), lambda i: (i, 0)),
        out_shape=jax.ShapeDtypeStruct((_NP, 128), F32),
    )(num1.reshape(2, _NP, 128), den1, b1.reshape(1, 256), Wl2, Wr2)

    num2, den2 = _sc_layer2(src2, dst2, wdst, xz, att2.reshape(64))
    num2 = num2.reshape(2, _N2R * 2, 64)[:, :_NP, :]
    den2 = den2.reshape(2, _DR * 8, 16)[:, :_NP, :]

    h2 = pl.pallas_call(
        _k5_body,
        grid=(grid_n,),
        in_specs=[pl.BlockSpec((2, bn, 64), lambda i: (0, i, 0)),
                  pl.BlockSpec((2, bn, 16), lambda i: (0, i, 0)),
                  pl.BlockSpec((1, 64), lambda i: (0, 0))],
        out_specs=pl.BlockSpec((bn, 128), lambda i: (i, 0)),
        out_shape=jax.ShapeDtypeStruct((_NP, 128), F32),
    )(num2, den2, b2.reshape(1, 64))

    hs, hd = _sc_qgather(qs, qd, h2)

    bq = 2048
    zz = pl.pallas_call(
        _k7_body,
        grid=(_EQP // bq,),
        in_specs=[pl.BlockSpec((bq, 128), lambda i: (i, 0)),
                  pl.BlockSpec((bq, 128), lambda i: (i, 0)),
                  pl.BlockSpec((bq, 16), lambda i: (i, 0)),
                  pl.BlockSpec((64, 128), lambda i: (0, 0)),
                  pl.BlockSpec((64, 128), lambda i: (0, 0)),
                  pl.BlockSpec((16, 128), lambda i: (0, 0)),
                  pl.BlockSpec((1, 128), lambda i: (0, 0)),
                  pl.BlockSpec((128, 64), lambda i: (0, 0)),
                  pl.BlockSpec((1, 64), lambda i: (0, 0)),
                  pl.BlockSpec((64, 1), lambda i: (0, 0)),
                  pl.BlockSpec((1, 1), lambda i: (0, 0))],
        out_specs=pl.BlockSpec((bq, 1), lambda i: (i, 0)),
        out_shape=jax.ShapeDtypeStruct((_EQP, 1), F32),
    )(hs, hd, qa, Wc1[0:64], Wc1[64:128], Wc1[128:144],
      bc1.reshape(1, 128), Wc2, bc2.reshape(1, 64), Wc3, bc3.reshape(1, 1))

    return zz[:_EQ, 0]
